# baseline (speedup 1.0000x reference)
import functools

import jax
import jax.numpy as jnp
from jax import lax
from jax.experimental import pallas as pl
from jax.experimental.pallas import tpu as pltpu
from jax.experimental.pallas import tpu_sc as plsc

N = 10000
E = 320000
NP = 10240
IN_DIM = 128
HID = 128
HID2 = 64
OUT = 7

NC = 2
NS = 16
NW = NC * NS
K = 80
NCH = 128
EPAD = NW * NCH * K
RPT = NP // NS
ZR = 160

EPS = 0.1
GAMMA = 0.1
NEG_SLOPE = 0.01

_mesh = functools.partial(
    plsc.VectorSubcoreMesh, core_axis_name="c", subcore_axis_name="s",
    num_cores=NC, num_subcores=NS)


def _zero_vmem(buf, rows, cols):
    lanes = cols // 16

    def body(i, carry):
        buf[i // lanes, pl.ds((i % lanes) * 16, 16)] = jnp.zeros((16,), jnp.float32)
        return carry

    lax.fori_loop(0, rows * lanes, body, 0)


def _make_deg_kernel():
    D = 16

    @functools.partial(
        pl.kernel,
        out_type=jax.ShapeDtypeStruct((NC, NP, D), jnp.float32),
        mesh=_mesh(),
        scratch_types=[
            pltpu.VMEM((K,), jnp.int32),
            pltpu.VMEM((K,), jnp.int32),
            pltpu.VMEM((K, D), jnp.float32),
            pltpu.VMEM((K, D), jnp.float32),
            pltpu.VMEM_SHARED((NP, D), jnp.float32),
            pltpu.SemaphoreType.DMA,
            pltpu.SemaphoreType.DMA,
        ],
    )
    def deg_kernel(dst_hbm, out_hbm, dbuf0, dbuf1, ones, zbuf, acc,
                   semd0, semd1):
        cid = lax.axis_index("c")
        sid = lax.axis_index("s")
        wid = sid * NC + cid

        def ones_body(i, carry):
            ones[i, pl.ds(0, 16)] = jnp.ones((16,), jnp.float32)
            return carry

        lax.fori_loop(0, K, ones_body, 0)

        _zero_vmem(zbuf, K, D)
        base_r = sid * RPT
        for b in range(RPT // K):
            pltpu.sync_copy(zbuf, acc.at[pl.ds(base_r + b * K, K)])
        plsc.subcore_barrier()

        def d_start(ch, dbuf, sem):
            pltpu.make_async_copy(dst_hbm.at[wid, ch], dbuf, sem).start()

        def d_wait(ch, dbuf, sem):
            pltpu.make_async_copy(dst_hbm.at[wid, ch], dbuf, sem).wait()

        d_start(0, dbuf0, semd0)

        def body(i, carry):
            ch0 = 2 * i
            ch1 = 2 * i + 1
            d_start(ch1, dbuf1, semd1)
            d_wait(ch0, dbuf0, semd0)
            pltpu.sync_copy(ones, acc.at[dbuf0], add=True)

            @pl.when(ch1 + 1 < NCH)
            def _():
                d_start(ch1 + 1, dbuf0, semd0)

            d_wait(ch1, dbuf1, semd1)
            pltpu.sync_copy(ones, acc.at[dbuf1], add=True)
            return carry

        lax.fori_loop(0, NCH // 2, body, 0)
        if NCH % 2:
            d_wait(NCH - 1, dbuf0, semd0)
            pltpu.sync_copy(ones, acc.at[dbuf0], add=True)
        plsc.subcore_barrier()

        for b in range(RPT // K):
            r = base_r + b * K
            pltpu.sync_copy(acc.at[pl.ds(r, K)], zbuf)
            pltpu.sync_copy(zbuf, out_hbm.at[cid, pl.ds(r, K)])

    return deg_kernel


def _make_agg_kernel(D):
    @functools.partial(
        pl.kernel,
        out_type=jax.ShapeDtypeStruct((NC, NP, D), jnp.float32),
        mesh=_mesh(),
        scratch_types=[
            [pltpu.VMEM((K,), jnp.int32)] * 3,
            [pltpu.VMEM((K,), jnp.int32)] * 3,
            [pltpu.VMEM((K, D), jnp.float32)] * 3,
            pltpu.VMEM_SHARED((NP, D), jnp.float32),
            [pltpu.SemaphoreType.DMA] * 3,
            [pltpu.SemaphoreType.DMA] * 3,
            [pltpu.SemaphoreType.DMA] * 3,
        ],
    )
    def agg_kernel(ms_hbm, src_hbm, dst_hbm, out_hbm,
                   sbuf, dbuf, rows, acc, semg, semi, semd):
        cid = lax.axis_index("c")
        sid = lax.axis_index("s")
        wid = sid * NC + cid
        rows0, rows1 = rows[0], rows[1]
        sem0, sem1 = semg[0], semg[1]

        _zero_vmem(rows0, K, D)
        base_r = sid * RPT
        for b in range(RPT // K):
            pltpu.sync_copy(rows0, acc.at[pl.ds(base_r + b * K, K)])
        plsc.subcore_barrier()

        def i_start(gch, s):
            pltpu.make_async_copy(src_hbm.at[wid, gch], sbuf[s], semi[s]).start()

        def i_wait(gch, s):
            pltpu.make_async_copy(src_hbm.at[wid, gch], sbuf[s], semi[s]).wait()

        def d_start(gch, s):
            pltpu.make_async_copy(dst_hbm.at[wid, gch], dbuf[s], semd[s]).start()

        def d_wait(gch, s):
            pltpu.make_async_copy(dst_hbm.at[wid, gch], dbuf[s], semd[s]).wait()

        def g_start(s):
            pltpu.make_async_copy(ms_hbm.at[sbuf[s]], rows[s], semg[s]).start()

        def g_wait(s):
            pltpu.make_async_copy(ms_hbm.at[sbuf[s]], rows[s], semg[s]).wait()

        def scat(s):
            pltpu.sync_copy(rows[s], acc.at[dbuf[s]], add=True)

        pltpu.sync_copy(src_hbm.at[wid, 0], sbuf[0])
        g_start(0)
        i_start(1, 1)
        i_start(2, 2)
        d_start(0, 0)
        d_start(1, 1)
        d_start(2, 2)
        i_wait(1, 1)
        g_start(1)

        def body(i, carry):
            for j in range(3):
                ch = 3 * i + j
                s = j
                s2 = (j + 2) % 3
                g_wait(s)

                @pl.when(ch + 2 < NCH)
                def _():
                    i_wait(ch + 2, s2)
                    g_start(s2)

                d_wait(ch, s)
                scat(s)

                @pl.when(ch + 3 < NCH)
                def _():
                    i_start(ch + 3, s)
                    d_start(ch + 3, s)

            return carry

        lax.fori_loop(0, NCH // 3, body, 0)
        for ch in range(3 * (NCH // 3), NCH):
            s = ch % 3
            g_wait(s)
            d_wait(ch, s)
            scat(s)
        plsc.subcore_barrier()

        nwo = RPT // K
        for b in range(nwo):
            buf = rows0 if b % 2 == 0 else rows1
            sem = sem0 if b % 2 == 0 else sem1
            if b >= 2:
                r_prev = base_r + (b - 2) * K
                pltpu.make_async_copy(
                    buf, out_hbm.at[cid, pl.ds(r_prev, K)], sem).wait()
            r = base_r + b * K
            pltpu.sync_copy(acc.at[pl.ds(r, K)], buf)
            pltpu.make_async_copy(buf, out_hbm.at[cid, pl.ds(r, K)], sem).start()
        for b in range(nwo - 2, nwo):
            buf = rows0 if b % 2 == 0 else rows1
            sem = sem0 if b % 2 == 0 else sem1
            r = base_r + b * K
            pltpu.make_async_copy(buf, out_hbm.at[cid, pl.ds(r, K)], sem).wait()

    return agg_kernel


_make_deg_kernel = functools.cache(_make_deg_kernel)
_make_agg_kernel = functools.cache(_make_agg_kernel)


def _deg_kernel(dst):
    return _make_deg_kernel()(dst)


def _agg128(ms, src, dst):
    return _make_agg_kernel(HID)(ms, src, dst)


BR = 512
GRID = NP // BR


def _leaky(x):
    return jnp.where(x >= 0, x, NEG_SLOPE * x)


def _dinv_from(degp_ref):
    deg = degp_ref[0, :, 0:1] + degp_ref[1, :, 0:1] + 1.0
    return lax.rsqrt(deg)


def _dotT(a, w):
    return lax.dot_general(a, w, (((1,), (1,)), ((), ())),
                           preferred_element_type=jnp.float32)


def _dot(a, w):
    return lax.dot_general(a, w, (((1,), (0,)), ((), ())),
                           preferred_element_type=jnp.float32)


def _row_spec(d):
    return pl.BlockSpec((BR, d), lambda i: (i, 0))


def _full_spec(r, c):
    return pl.BlockSpec((r, c), lambda i: (0, 0))


_degp_spec = pl.BlockSpec((NC, BR, 16), lambda i: (0, i, 0))
_accp_spec128 = pl.BlockSpec((NC, BR, HID), lambda i: (0, i, 0))
_accp_spec64 = pl.BlockSpec((NC, BR, HID2), lambda i: (0, i, 0))


def _k1_body(x_ref, w1_ref, b1_ref, gw_ref, degp_ref,
             h_ref, m_ref, ms_ref):
    dinv = _dinv_from(degp_ref)
    h = _leaky(_dot(x_ref[...], w1_ref[...]) + b1_ref[...])
    m = _dot(h, gw_ref[...])
    h_ref[...] = h
    m_ref[...] = m
    ms_ref[...] = m * dinv


def _iter_update(h, m, accp_ref, dinv, asw_ref, b_ref):
    acc = accp_ref[0] + accp_ref[1]
    g = dinv * acc + (dinv * dinv) * m
    z = _dotT(h, asw_ref[...]) - _dot(h, asw_ref[...]) - GAMMA * h + g + b_ref[...]
    return h + EPS * jnp.tanh(z)


def _k2_body(h_ref, m_ref, accp_ref, degp_ref, asw_ref, b_ref, gw_ref,
             h_out, m_out, ms_out):
    dinv = _dinv_from(degp_ref)
    h2 = _iter_update(h_ref[...], m_ref[...], accp_ref, dinv, asw_ref, b_ref)
    m2 = _dot(h2, gw_ref[...])
    h_out[...] = h2
    m_out[...] = m2
    ms_out[...] = m2 * dinv


def _k4_body(h_ref, m_ref, accp_ref, degp_ref, asw_ref, b_ref,
             w2_ref, b2_ref, gw2_ref,
             h_out, m_out, ms_out):
    dinv = _dinv_from(degp_ref)
    h2 = _iter_update(h_ref[...], m_ref[...], accp_ref, dinv, asw_ref, b_ref)
    hb = _leaky(_dot(_leaky(h2), w2_ref[...]) + b2_ref[...])
    mb = _dot(hb, gw2_ref[...])
    h_out[...] = hb
    m_out[...] = mb
    ms_out[...] = mb * dinv


def _k5_body(h_ref, m_ref, accp_ref, degp_ref, asw_ref, b_ref,
             wf_ref, bf_ref, out_ref):
    dinv = _dinv_from(degp_ref)
    h2 = _iter_update(h_ref[...], m_ref[...], accp_ref, dinv, asw_ref, b_ref)
    logits = _dot(h2, wf_ref[...]) + bf_ref[...]
    col = lax.broadcasted_iota(jnp.int32, logits.shape, 1)
    z = jnp.where(col < OUT, logits, -1e30)
    zmax = jnp.max(z, axis=1, keepdims=True)
    lse = jnp.log(jnp.sum(jnp.exp(z - zmax), axis=1, keepdims=True)) + zmax
    out_ref[...] = z - lse


def _rows_out(d):
    return jax.ShapeDtypeStruct((NP, d), jnp.float32)


_k1 = pl.pallas_call(
    _k1_body,
    grid=(GRID,),
    in_specs=[_row_spec(IN_DIM), _full_spec(IN_DIM, HID), _full_spec(1, HID),
              _full_spec(HID, HID), _degp_spec],
    out_specs=[_row_spec(HID)] * 3,
    out_shape=[_rows_out(HID)] * 3,
)

_k2 = pl.pallas_call(
    _k2_body,
    grid=(GRID,),
    in_specs=[_row_spec(HID), _row_spec(HID), _accp_spec128, _degp_spec,
              _full_spec(HID, HID), _full_spec(1, HID), _full_spec(HID, HID)],
    out_specs=[_row_spec(HID)] * 3,
    out_shape=[_rows_out(HID)] * 3,
)

_k4 = pl.pallas_call(
    _k4_body,
    grid=(GRID,),
    in_specs=[_row_spec(HID), _row_spec(HID), _accp_spec128, _degp_spec,
              _full_spec(HID, HID), _full_spec(1, HID),
              _full_spec(HID, HID), _full_spec(1, HID), _full_spec(HID, HID)],
    out_specs=[_row_spec(HID)] * 3,
    out_shape=[_rows_out(HID)] * 3,
)

_k5 = pl.pallas_call(
    _k5_body,
    grid=(GRID,),
    in_specs=[_row_spec(HID), _row_spec(HID), _accp_spec128, _degp_spec,
              _full_spec(HID, HID), _full_spec(1, HID),
              _full_spec(HID, 128), _full_spec(1, 128)],
    out_specs=_row_spec(128),
    out_shape=jax.ShapeDtypeStruct((NP, 128), jnp.float32),
)


def kernel(x, edge_index, W1, b1, asW1, asb1, gW1, W2, b2, asW2, asb2, gW2, Wf, bf):
    pad = (jnp.arange(EPAD - E, dtype=edge_index.dtype) % (NP - N)) + N
    src = jnp.concatenate([edge_index[0], pad]).reshape(NW, NCH, K)
    dst = jnp.concatenate([edge_index[1], pad]).reshape(NW, NCH, K)
    xp = jnp.pad(x, ((0, NP - N), (0, 0)))
    b1r = b1.reshape(1, HID)
    asb1r = asb1.reshape(1, HID)
    w2p = jnp.pad(W2, ((0, 0), (0, HID - HID2)))
    b2p = jnp.pad(b2, ((0, HID - HID2),)).reshape(1, HID)
    asw2p = jnp.pad(asW2, ((0, HID - HID2), (0, HID - HID2)))
    asb2p = jnp.pad(asb2, ((0, HID - HID2),)).reshape(1, HID)
    gw2p = jnp.pad(gW2, ((0, HID - HID2), (0, HID - HID2)))
    wfp = jnp.pad(Wf, ((0, HID - HID2), (0, 128 - OUT)))
    bfp = jnp.pad(bf, ((0, 128 - OUT),)).reshape(1, 128)

    degp = _deg_kernel(dst)

    h, m, ms = _k1(xp, W1, b1r, gW1, degp)
    for _ in range(2):
        accp = _agg128(ms, src, dst)
        h, m, ms = _k2(h, m, accp, degp, asW1, asb1r, gW1)
    accp = _agg128(ms, src, dst)
    h, m, ms = _k4(h, m, accp, degp, asW1, asb1r, w2p, b2p, gw2p)
    accp = _agg128(ms, src, dst)
    out = _k5(h, m, accp, degp, asw2p, asb2p, wfp, bfp)
    return out[:N, :OUT]


# speedup vs baseline: 1.2662x; 1.2662x over previous
import functools

import jax
import jax.numpy as jnp
from jax import lax
from jax.experimental import pallas as pl
from jax.experimental.pallas import tpu as pltpu
from jax.experimental.pallas import tpu_sc as plsc

N = 10000
E = 320000
NP = 10240
IN_DIM = 128
HID = 128
HID2 = 64
OUT = 7

NC = 2
NS = 16
NW = NC * NS
K = 80
NCH = 128
EPAD = NW * NCH * K
RPT = NP // NS
ZR = 160

EPS = 0.1
GAMMA = 0.1
NEG_SLOPE = 0.01

_mesh = functools.partial(
    plsc.VectorSubcoreMesh, core_axis_name="c", subcore_axis_name="s",
    num_cores=NC, num_subcores=NS)


def _zero_vmem(buf, rows, cols):
    lanes = cols // 16

    def body(i, carry):
        buf[i // lanes, pl.ds((i % lanes) * 16, 16)] = jnp.zeros((16,), jnp.float32)
        return carry

    lax.fori_loop(0, rows * lanes, body, 0)


def _make_deg_kernel():
    D = 16

    @functools.partial(
        pl.kernel,
        out_type=jax.ShapeDtypeStruct((NC, NP, D), jnp.float32),
        mesh=_mesh(),
        scratch_types=[
            pltpu.VMEM((K,), jnp.int32),
            pltpu.VMEM((K,), jnp.int32),
            pltpu.VMEM((K, D), jnp.float32),
            pltpu.VMEM((K, D), jnp.float32),
            pltpu.VMEM_SHARED((NP, D), jnp.float32),
            pltpu.SemaphoreType.DMA,
            pltpu.SemaphoreType.DMA,
        ],
    )
    def deg_kernel(dst_hbm, out_hbm, dbuf0, dbuf1, ones, zbuf, acc,
                   semd0, semd1):
        cid = lax.axis_index("c")
        sid = lax.axis_index("s")
        wid = sid * NC + cid

        def ones_body(i, carry):
            ones[i, pl.ds(0, 16)] = jnp.ones((16,), jnp.float32)
            return carry

        lax.fori_loop(0, K, ones_body, 0)

        _zero_vmem(zbuf, K, D)
        base_r = sid * RPT
        for b in range(RPT // K):
            pltpu.sync_copy(zbuf, acc.at[pl.ds(base_r + b * K, K)])
        plsc.subcore_barrier()

        def d_start(ch, dbuf, sem):
            pltpu.make_async_copy(dst_hbm.at[wid, ch], dbuf, sem).start()

        def d_wait(ch, dbuf, sem):
            pltpu.make_async_copy(dst_hbm.at[wid, ch], dbuf, sem).wait()

        d_start(0, dbuf0, semd0)

        def body(i, carry):
            ch0 = 2 * i
            ch1 = 2 * i + 1
            d_start(ch1, dbuf1, semd1)
            d_wait(ch0, dbuf0, semd0)
            pltpu.sync_copy(ones, acc.at[dbuf0], add=True)

            @pl.when(ch1 + 1 < NCH)
            def _():
                d_start(ch1 + 1, dbuf0, semd0)

            d_wait(ch1, dbuf1, semd1)
            pltpu.sync_copy(ones, acc.at[dbuf1], add=True)
            return carry

        lax.fori_loop(0, NCH // 2, body, 0)
        if NCH % 2:
            d_wait(NCH - 1, dbuf0, semd0)
            pltpu.sync_copy(ones, acc.at[dbuf0], add=True)
        plsc.subcore_barrier()

        for b in range(RPT // K):
            r = base_r + b * K
            pltpu.sync_copy(acc.at[pl.ds(r, K)], zbuf)
            pltpu.sync_copy(zbuf, out_hbm.at[cid, pl.ds(r, K)])

    return deg_kernel


def _make_agg_kernel(D):
    @functools.partial(
        pl.kernel,
        out_type=jax.ShapeDtypeStruct((NC, NP, D), jnp.float32),
        mesh=_mesh(),
        scratch_types=[
            [pltpu.VMEM((K,), jnp.int32)] * 6,
            [pltpu.VMEM((K,), jnp.int32)] * 6,
            [pltpu.VMEM((K, D), jnp.float32)] * 3,
            pltpu.VMEM_SHARED((NP, D), jnp.float32),
            [pltpu.SemaphoreType.DMA] * 3,
            [pltpu.SemaphoreType.DMA] * 3,
            [pltpu.SemaphoreType.DMA] * 6,
            [pltpu.SemaphoreType.DMA] * 6,
        ],
    )
    def agg_kernel(ms_hbm, src_hbm, dst_hbm, out_hbm,
                   sbuf, dbuf, rows, acc, semg, sems, semi, semd):
        cid = lax.axis_index("c")
        sid = lax.axis_index("s")
        wid = sid * NC + cid
        rows0, rows1 = rows[0], rows[1]
        sem0, sem1 = semg[0], semg[1]

        _zero_vmem(rows0, K, D)
        base_r = sid * RPT
        for b in range(RPT // K):
            pltpu.sync_copy(rows0, acc.at[pl.ds(base_r + b * K, K)])
        plsc.subcore_barrier()

        def i_start(gch, m):
            pltpu.make_async_copy(src_hbm.at[wid, gch], sbuf[m], semi[m]).start()

        def i_wait(gch, m):
            pltpu.make_async_copy(src_hbm.at[wid, gch], sbuf[m], semi[m]).wait()

        def d_start(gch, m):
            pltpu.make_async_copy(dst_hbm.at[wid, gch], dbuf[m], semd[m]).start()

        def d_wait(gch, m):
            pltpu.make_async_copy(dst_hbm.at[wid, gch], dbuf[m], semd[m]).wait()

        def g_start(m, u):
            pltpu.make_async_copy(ms_hbm.at[sbuf[m]], rows[u], semg[u]).start()

        def g_wait(m, u):
            pltpu.make_async_copy(ms_hbm.at[sbuf[m]], rows[u], semg[u]).wait()

        def scat_start(u, m):
            pltpu.make_async_copy(rows[u], acc.at[dbuf[m]], sems[u]).start()

        def scat_wait(u, m):
            pltpu.make_async_copy(rows[u], acc.at[dbuf[m]], sems[u]).wait()

        for c in range(4):
            i_start(c, c)
            d_start(c, c)
        i_wait(0, 0)
        g_start(0, 0)
        i_wait(1, 1)
        g_start(1, 1)

        def step(t, j, static=False):
            def maybe(cond, fn):
                if static:
                    if cond:
                        fn()
                else:
                    pl.when(cond)(fn)

            u = j % 3
            g_wait(j, u)
            d_wait(t, j)
            scat_start(u, j)

            j1 = (j + 5) % 6
            maybe((t >= 1) & (t + 2 < NCH), lambda: scat_wait(j1 % 3, j1))

            j2 = (j + 2) % 6

            def adv_gather():
                i_wait(t + 2, j2)
                g_start(j2, j2 % 3)

            maybe(t + 2 < NCH, adv_gather)

            j4 = (j + 4) % 6

            def adv_idx():
                i_start(t + 4, j4)
                d_start(t + 4, j4)

            maybe(t + 4 < NCH, adv_idx)

        def body(i, carry):
            for j in range(6):
                step(6 * i + j, j)
            return carry

        NB = (NCH - 2) // 6
        lax.fori_loop(0, NB, body, 0)
        for t in range(6 * NB, NCH):
            step(t, t % 6, static=True)
        for c in range(NCH - 3, NCH):
            scat_wait(c % 3, c % 6)
        plsc.subcore_barrier()

        nwo = RPT // K
        for b in range(nwo):
            buf = rows0 if b % 2 == 0 else rows1
            sem = sem0 if b % 2 == 0 else sem1
            if b >= 2:
                r_prev = base_r + (b - 2) * K
                pltpu.make_async_copy(
                    buf, out_hbm.at[cid, pl.ds(r_prev, K)], sem).wait()
            r = base_r + b * K
            pltpu.sync_copy(acc.at[pl.ds(r, K)], buf)
            pltpu.make_async_copy(buf, out_hbm.at[cid, pl.ds(r, K)], sem).start()
        for b in range(nwo - 2, nwo):
            buf = rows0 if b % 2 == 0 else rows1
            sem = sem0 if b % 2 == 0 else sem1
            r = base_r + b * K
            pltpu.make_async_copy(buf, out_hbm.at[cid, pl.ds(r, K)], sem).wait()

    return agg_kernel


_make_deg_kernel = functools.cache(_make_deg_kernel)
_make_agg_kernel = functools.cache(_make_agg_kernel)


def _deg_kernel(dst):
    return _make_deg_kernel()(dst)


def _agg128(ms, src, dst):
    return _make_agg_kernel(HID)(ms, src, dst)


BR = 512
GRID = NP // BR


def _leaky(x):
    return jnp.where(x >= 0, x, NEG_SLOPE * x)


def _dinv_from(degp_ref):
    deg = degp_ref[0, :, 0:1] + degp_ref[1, :, 0:1] + 1.0
    return lax.rsqrt(deg)


def _dotT(a, w):
    return lax.dot_general(a, w, (((1,), (1,)), ((), ())),
                           preferred_element_type=jnp.float32)


def _dot(a, w):
    return lax.dot_general(a, w, (((1,), (0,)), ((), ())),
                           preferred_element_type=jnp.float32)


def _row_spec(d):
    return pl.BlockSpec((BR, d), lambda i: (i, 0))


def _full_spec(r, c):
    return pl.BlockSpec((r, c), lambda i: (0, 0))


_degp_spec = pl.BlockSpec((NC, BR, 16), lambda i: (0, i, 0))
_accp_spec128 = pl.BlockSpec((NC, BR, HID), lambda i: (0, i, 0))
_accp_spec64 = pl.BlockSpec((NC, BR, HID2), lambda i: (0, i, 0))


def _k1_body(x_ref, w1_ref, b1_ref, gw_ref, degp_ref,
             h_ref, m_ref, ms_ref):
    dinv = _dinv_from(degp_ref)
    h = _leaky(_dot(x_ref[...], w1_ref[...]) + b1_ref[...])
    m = _dot(h, gw_ref[...])
    h_ref[...] = h
    m_ref[...] = m
    ms_ref[...] = m * dinv


def _iter_update(h, m, accp_ref, dinv, asw_ref, b_ref):
    acc = accp_ref[0] + accp_ref[1]
    g = dinv * acc + (dinv * dinv) * m
    z = _dotT(h, asw_ref[...]) - _dot(h, asw_ref[...]) - GAMMA * h + g + b_ref[...]
    return h + EPS * jnp.tanh(z)


def _k2_body(h_ref, m_ref, accp_ref, degp_ref, asw_ref, b_ref, gw_ref,
             h_out, m_out, ms_out):
    dinv = _dinv_from(degp_ref)
    h2 = _iter_update(h_ref[...], m_ref[...], accp_ref, dinv, asw_ref, b_ref)
    m2 = _dot(h2, gw_ref[...])
    h_out[...] = h2
    m_out[...] = m2
    ms_out[...] = m2 * dinv


def _k4_body(h_ref, m_ref, accp_ref, degp_ref, asw_ref, b_ref,
             w2_ref, b2_ref, gw2_ref,
             h_out, m_out, ms_out):
    dinv = _dinv_from(degp_ref)
    h2 = _iter_update(h_ref[...], m_ref[...], accp_ref, dinv, asw_ref, b_ref)
    hb = _leaky(_dot(_leaky(h2), w2_ref[...]) + b2_ref[...])
    mb = _dot(hb, gw2_ref[...])
    h_out[...] = hb
    m_out[...] = mb
    ms_out[...] = mb * dinv


def _k5_body(h_ref, m_ref, accp_ref, degp_ref, asw_ref, b_ref,
             wf_ref, bf_ref, out_ref):
    dinv = _dinv_from(degp_ref)
    h2 = _iter_update(h_ref[...], m_ref[...], accp_ref, dinv, asw_ref, b_ref)
    logits = _dot(h2, wf_ref[...]) + bf_ref[...]
    col = lax.broadcasted_iota(jnp.int32, logits.shape, 1)
    z = jnp.where(col < OUT, logits, -1e30)
    zmax = jnp.max(z, axis=1, keepdims=True)
    lse = jnp.log(jnp.sum(jnp.exp(z - zmax), axis=1, keepdims=True)) + zmax
    out_ref[...] = z - lse


def _rows_out(d):
    return jax.ShapeDtypeStruct((NP, d), jnp.float32)


_k1 = pl.pallas_call(
    _k1_body,
    grid=(GRID,),
    in_specs=[_row_spec(IN_DIM), _full_spec(IN_DIM, HID), _full_spec(1, HID),
              _full_spec(HID, HID), _degp_spec],
    out_specs=[_row_spec(HID)] * 3,
    out_shape=[_rows_out(HID)] * 3,
)

_k2 = pl.pallas_call(
    _k2_body,
    grid=(GRID,),
    in_specs=[_row_spec(HID), _row_spec(HID), _accp_spec128, _degp_spec,
              _full_spec(HID, HID), _full_spec(1, HID), _full_spec(HID, HID)],
    out_specs=[_row_spec(HID)] * 3,
    out_shape=[_rows_out(HID)] * 3,
)

_k4 = pl.pallas_call(
    _k4_body,
    grid=(GRID,),
    in_specs=[_row_spec(HID), _row_spec(HID), _accp_spec128, _degp_spec,
              _full_spec(HID, HID), _full_spec(1, HID),
              _full_spec(HID, HID), _full_spec(1, HID), _full_spec(HID, HID)],
    out_specs=[_row_spec(HID)] * 3,
    out_shape=[_rows_out(HID)] * 3,
)

_k5 = pl.pallas_call(
    _k5_body,
    grid=(GRID,),
    in_specs=[_row_spec(HID), _row_spec(HID), _accp_spec128, _degp_spec,
              _full_spec(HID, HID), _full_spec(1, HID),
              _full_spec(HID, 128), _full_spec(1, 128)],
    out_specs=_row_spec(128),
    out_shape=jax.ShapeDtypeStruct((NP, 128), jnp.float32),
)


def kernel(x, edge_index, W1, b1, asW1, asb1, gW1, W2, b2, asW2, asb2, gW2, Wf, bf):
    pad = (jnp.arange(EPAD - E, dtype=edge_index.dtype) % (NP - N)) + N
    src = jnp.concatenate([edge_index[0], pad]).reshape(NW, NCH, K)
    dst = jnp.concatenate([edge_index[1], pad]).reshape(NW, NCH, K)
    xp = jnp.pad(x, ((0, NP - N), (0, 0)))
    b1r = b1.reshape(1, HID)
    asb1r = asb1.reshape(1, HID)
    w2p = jnp.pad(W2, ((0, 0), (0, HID - HID2)))
    b2p = jnp.pad(b2, ((0, HID - HID2),)).reshape(1, HID)
    asw2p = jnp.pad(asW2, ((0, HID - HID2), (0, HID - HID2)))
    asb2p = jnp.pad(asb2, ((0, HID - HID2),)).reshape(1, HID)
    gw2p = jnp.pad(gW2, ((0, HID - HID2), (0, HID - HID2)))
    wfp = jnp.pad(Wf, ((0, HID - HID2), (0, 128 - OUT)))
    bfp = jnp.pad(bf, ((0, 128 - OUT),)).reshape(1, 128)

    degp = _deg_kernel(dst)

    h, m, ms = _k1(xp, W1, b1r, gW1, degp)
    for _ in range(2):
        accp = _agg128(ms, src, dst)
        h, m, ms = _k2(h, m, accp, degp, asW1, asb1r, gW1)
    accp = _agg128(ms, src, dst)
    h, m, ms = _k4(h, m, accp, degp, asW1, asb1r, w2p, b2p, gw2p)
    accp = _agg128(ms, src, dst)
    out = _k5(h, m, accp, degp, asw2p, asb2p, wfp, bfp)
    return out[:N, :OUT]


# speedup vs baseline: 1.2885x; 1.0176x over previous
import functools

import jax
import jax.numpy as jnp
from jax import lax
from jax.experimental import pallas as pl
from jax.experimental.pallas import tpu as pltpu
from jax.experimental.pallas import tpu_sc as plsc

N = 10000
E = 320000
NP = 10240
IN_DIM = 128
HID = 128
HID2 = 64
OUT = 7

NC = 2
NS = 16
NW = NC * NS
K = 80
NCH = 128
EPAD = NW * NCH * K
RPT = NP // NS
ZR = 160

EPS = 0.1
GAMMA = 0.1
NEG_SLOPE = 0.01

_mesh = functools.partial(
    plsc.VectorSubcoreMesh, core_axis_name="c", subcore_axis_name="s",
    num_cores=NC, num_subcores=NS)


def _zero_vmem(buf, rows, cols):
    lanes = cols // 16

    def body(i, carry):
        buf[i // lanes, pl.ds((i % lanes) * 16, 16)] = jnp.zeros((16,), jnp.float32)
        return carry

    lax.fori_loop(0, rows * lanes, body, 0)


def _make_deg_kernel():
    D = 16

    @functools.partial(
        pl.kernel,
        out_type=jax.ShapeDtypeStruct((NC, NP, D), jnp.float32),
        mesh=_mesh(),
        scratch_types=[
            pltpu.VMEM((K,), jnp.int32),
            pltpu.VMEM((K,), jnp.int32),
            pltpu.VMEM((K, D), jnp.float32),
            pltpu.VMEM((K, D), jnp.float32),
            pltpu.VMEM_SHARED((NP, D), jnp.float32),
            pltpu.SemaphoreType.DMA,
            pltpu.SemaphoreType.DMA,
        ],
    )
    def deg_kernel(dst_hbm, out_hbm, dbuf0, dbuf1, ones, zbuf, acc,
                   semd0, semd1):
        cid = lax.axis_index("c")
        sid = lax.axis_index("s")
        wid = sid * NC + cid

        def ones_body(i, carry):
            ones[i, pl.ds(0, 16)] = jnp.ones((16,), jnp.float32)
            return carry

        lax.fori_loop(0, K, ones_body, 0)

        _zero_vmem(zbuf, K, D)
        base_r = sid * RPT
        for b in range(RPT // K):
            pltpu.sync_copy(zbuf, acc.at[pl.ds(base_r + b * K, K)])
        plsc.subcore_barrier()

        def d_start(ch, dbuf, sem):
            pltpu.make_async_copy(dst_hbm.at[wid, ch], dbuf, sem).start()

        def d_wait(ch, dbuf, sem):
            pltpu.make_async_copy(dst_hbm.at[wid, ch], dbuf, sem).wait()

        d_start(0, dbuf0, semd0)

        def body(i, carry):
            ch0 = 2 * i
            ch1 = 2 * i + 1
            d_start(ch1, dbuf1, semd1)
            d_wait(ch0, dbuf0, semd0)
            pltpu.sync_copy(ones, acc.at[dbuf0], add=True)

            @pl.when(ch1 + 1 < NCH)
            def _():
                d_start(ch1 + 1, dbuf0, semd0)

            d_wait(ch1, dbuf1, semd1)
            pltpu.sync_copy(ones, acc.at[dbuf1], add=True)
            return carry

        lax.fori_loop(0, NCH // 2, body, 0)
        if NCH % 2:
            d_wait(NCH - 1, dbuf0, semd0)
            pltpu.sync_copy(ones, acc.at[dbuf0], add=True)
        plsc.subcore_barrier()

        for b in range(RPT // K):
            r = base_r + b * K
            pltpu.sync_copy(acc.at[pl.ds(r, K)], zbuf)
            pltpu.sync_copy(zbuf, out_hbm.at[cid, pl.ds(r, K)])

    return deg_kernel


def _make_agg_kernel(D):
    @functools.partial(
        pl.kernel,
        out_type=jax.ShapeDtypeStruct((NC, NP, D), jnp.float32),
        mesh=_mesh(),
        scratch_types=[
            [pltpu.VMEM((K,), jnp.int32)] * 6,
            [pltpu.VMEM((K,), jnp.int32)] * 6,
            [pltpu.VMEM((K, D), jnp.float32)] * 3,
            pltpu.VMEM_SHARED((NP, D), jnp.float32),
            [pltpu.SemaphoreType.DMA] * 3,
            [pltpu.SemaphoreType.DMA] * 3,
            [pltpu.SemaphoreType.DMA] * 6,
            [pltpu.SemaphoreType.DMA] * 6,
        ],
    )
    def agg_kernel(ms_hbm, src_hbm, dst_hbm, out_hbm,
                   sbuf, dbuf, rows, acc, semg, sems, semi, semd):
        cid = lax.axis_index("c")
        sid = lax.axis_index("s")
        wid = sid * NC + cid
        rows0, rows1 = rows[0], rows[1]
        sem0, sem1 = semg[0], semg[1]

        _zero_vmem(rows0, K, D)
        base_r = sid * RPT
        for b in range(RPT // K):
            pltpu.sync_copy(rows0, acc.at[pl.ds(base_r + b * K, K)])
        plsc.subcore_barrier()

        def i_start(gch, m):
            pltpu.make_async_copy(src_hbm.at[wid, gch], sbuf[m], semi[m]).start()

        def i_wait(gch, m):
            pltpu.make_async_copy(src_hbm.at[wid, gch], sbuf[m], semi[m]).wait()

        def d_start(gch, m):
            pltpu.make_async_copy(dst_hbm.at[wid, gch], dbuf[m], semd[m]).start()

        def d_wait(gch, m):
            pltpu.make_async_copy(dst_hbm.at[wid, gch], dbuf[m], semd[m]).wait()

        def g_start(m, u):
            pltpu.make_async_copy(ms_hbm.at[sbuf[m]], rows[u], semg[u]).start()

        def g_wait(m, u):
            pltpu.make_async_copy(ms_hbm.at[sbuf[m]], rows[u], semg[u]).wait()

        def scat_start(u, m):
            pltpu.make_async_copy(rows[u], acc.at[dbuf[m]], sems[u]).start()

        def scat_wait(u, m):
            pltpu.make_async_copy(rows[u], acc.at[dbuf[m]], sems[u]).wait()

        for c in range(4):
            i_start(c, c)
            d_start(c, c)
        i_wait(0, 0)
        g_start(0, 0)
        i_wait(1, 1)
        g_start(1, 1)

        def step(t, j, static=False):
            def maybe(cond, fn):
                if static:
                    if cond:
                        fn()
                else:
                    pl.when(cond)(fn)

            u = j % 3
            g_wait(j, u)
            d_wait(t, j)
            scat_start(u, j)

            j1 = (j + 5) % 6
            maybe((t >= 1) & (t + 2 < NCH), lambda: scat_wait(j1 % 3, j1))

            j2 = (j + 2) % 6

            def adv_gather():
                i_wait(t + 2, j2)
                g_start(j2, j2 % 3)

            maybe(t + 2 < NCH, adv_gather)

            j4 = (j + 4) % 6

            def adv_idx():
                i_start(t + 4, j4)
                d_start(t + 4, j4)

            maybe(t + 4 < NCH, adv_idx)

        def body(i, carry):
            for j in range(6):
                step(6 * i + j, j)
            return carry

        NB = (NCH - 2) // 6
        lax.fori_loop(0, NB, body, 0)
        for t in range(6 * NB, NCH):
            step(t, t % 6, static=True)
        for c in range(NCH - 3, NCH):
            scat_wait(c % 3, c % 6)
        plsc.subcore_barrier()

        nwo = RPT // K
        for b in range(nwo):
            buf = rows0 if b % 2 == 0 else rows1
            sem = sem0 if b % 2 == 0 else sem1
            if b >= 2:
                r_prev = base_r + (b - 2) * K
                pltpu.make_async_copy(
                    buf, out_hbm.at[cid, pl.ds(r_prev, K)], sem).wait()
            r = base_r + b * K
            pltpu.sync_copy(acc.at[pl.ds(r, K)], buf)
            pltpu.make_async_copy(buf, out_hbm.at[cid, pl.ds(r, K)], sem).start()
        for b in range(nwo - 2, nwo):
            buf = rows0 if b % 2 == 0 else rows1
            sem = sem0 if b % 2 == 0 else sem1
            r = base_r + b * K
            pltpu.make_async_copy(buf, out_hbm.at[cid, pl.ds(r, K)], sem).wait()

    return agg_kernel


_make_deg_kernel = functools.cache(_make_deg_kernel)
_make_agg_kernel = functools.cache(_make_agg_kernel)


def _deg_kernel(dst):
    return _make_deg_kernel()(dst)


def _agg128(ms, src, dst):
    return _make_agg_kernel(HID)(ms, src, dst)


BR = 512
GRID = NP // BR


def _leaky(x):
    return jnp.where(x >= 0, x, NEG_SLOPE * x)


def _dinv_from(degp_ref):
    deg = degp_ref[0, :, 0:1] + degp_ref[1, :, 0:1] + 1.0
    return lax.rsqrt(deg)


def _dotT(a, w):
    return lax.dot_general(a, w, (((1,), (1,)), ((), ())),
                           preferred_element_type=jnp.float32)


def _dot(a, w):
    return lax.dot_general(a, w, (((1,), (0,)), ((), ())),
                           preferred_element_type=jnp.float32)


def _row_spec(d):
    return pl.BlockSpec((BR, d), lambda i: (i, 0))


def _full_spec(r, c):
    return pl.BlockSpec((r, c), lambda i: (0, 0))


_degp_spec = pl.BlockSpec((NC, BR, 16), lambda i: (0, i, 0))
_accp_spec128 = pl.BlockSpec((NC, BR, HID), lambda i: (0, i, 0))
_accp_spec64 = pl.BlockSpec((NC, BR, HID2), lambda i: (0, i, 0))


def _k1a_body(x_ref, w1_ref, b1_ref, h_ref):
    h_ref[...] = _leaky(_dot(x_ref[...], w1_ref[...]) + b1_ref[...])


def _k1b_body(h_ref, gw_ref, degp_ref, ms_ref, dinv_ref):
    dinv = _dinv_from(degp_ref)
    ms_ref[...] = _dot(h_ref[...], gw_ref[...]) * dinv
    dinv_ref[...] = dinv


def _iter_update(h, accp_ref, dinv, asw_ref, b_ref, gw_ref):
    m = _dot(h, gw_ref[...])
    acc = accp_ref[0] + accp_ref[1]
    g = dinv * acc + (dinv * dinv) * m
    z = _dotT(h, asw_ref[...]) - _dot(h, asw_ref[...]) - GAMMA * h + g + b_ref[...]
    return h + EPS * jnp.tanh(z)


def _k2_body(h_ref, accp_ref, dinv_ref, asw_ref, b_ref, gw_ref,
             h_out, ms_out):
    dinv = dinv_ref[...]
    h2 = _iter_update(h_ref[...], accp_ref, dinv, asw_ref, b_ref, gw_ref)
    h_out[...] = h2
    ms_out[...] = _dot(h2, gw_ref[...]) * dinv


def _k4_body(h_ref, accp_ref, dinv_ref, asw_ref, b_ref, gw_ref,
             w2_ref, b2_ref, gw2_ref,
             h_out, ms_out):
    dinv = dinv_ref[...]
    h2 = _iter_update(h_ref[...], accp_ref, dinv, asw_ref, b_ref, gw_ref)
    hb = _leaky(_dot(_leaky(h2), w2_ref[...]) + b2_ref[...])
    h_out[...] = hb
    ms_out[...] = _dot(hb, gw2_ref[...]) * dinv


def _k5_body(h_ref, accp_ref, dinv_ref, asw_ref, b_ref, gw_ref,
             wf_ref, bf_ref, out_ref):
    dinv = dinv_ref[...]
    h2 = _iter_update(h_ref[...], accp_ref, dinv, asw_ref, b_ref, gw_ref)
    logits = _dot(h2, wf_ref[...]) + bf_ref[...]
    col = lax.broadcasted_iota(jnp.int32, logits.shape, 1)
    z = jnp.where(col < OUT, logits, -1e30)
    zmax = jnp.max(z, axis=1, keepdims=True)
    lse = jnp.log(jnp.sum(jnp.exp(z - zmax), axis=1, keepdims=True)) + zmax
    out_ref[...] = z - lse


def _rows_out(d):
    return jax.ShapeDtypeStruct((NP, d), jnp.float32)


_dinv_spec = pl.BlockSpec((BR, 1), lambda i: (i, 0))

_k1a = pl.pallas_call(
    _k1a_body,
    grid=(GRID,),
    in_specs=[_row_spec(IN_DIM), _full_spec(IN_DIM, HID), _full_spec(1, HID)],
    out_specs=_row_spec(HID),
    out_shape=_rows_out(HID),
)

_k1b = pl.pallas_call(
    _k1b_body,
    grid=(GRID,),
    in_specs=[_row_spec(HID), _full_spec(HID, HID), _degp_spec],
    out_specs=[_row_spec(HID), _dinv_spec],
    out_shape=[_rows_out(HID), jax.ShapeDtypeStruct((NP, 1), jnp.float32)],
)

_k2 = pl.pallas_call(
    _k2_body,
    grid=(GRID,),
    in_specs=[_row_spec(HID), _accp_spec128, _dinv_spec,
              _full_spec(HID, HID), _full_spec(1, HID), _full_spec(HID, HID)],
    out_specs=[_row_spec(HID)] * 2,
    out_shape=[_rows_out(HID)] * 2,
)

_k4 = pl.pallas_call(
    _k4_body,
    grid=(GRID,),
    in_specs=[_row_spec(HID), _accp_spec128, _dinv_spec,
              _full_spec(HID, HID), _full_spec(1, HID), _full_spec(HID, HID),
              _full_spec(HID, HID), _full_spec(1, HID), _full_spec(HID, HID)],
    out_specs=[_row_spec(HID)] * 2,
    out_shape=[_rows_out(HID)] * 2,
)

_k5 = pl.pallas_call(
    _k5_body,
    grid=(GRID,),
    in_specs=[_row_spec(HID), _accp_spec128, _dinv_spec,
              _full_spec(HID, HID), _full_spec(1, HID), _full_spec(HID, HID),
              _full_spec(HID, 128), _full_spec(1, 128)],
    out_specs=_row_spec(128),
    out_shape=jax.ShapeDtypeStruct((NP, 128), jnp.float32),
)


def kernel(x, edge_index, W1, b1, asW1, asb1, gW1, W2, b2, asW2, asb2, gW2, Wf, bf):
    pad = (jnp.arange(EPAD - E, dtype=edge_index.dtype) % (NP - N)) + N
    src = jnp.concatenate([edge_index[0], pad]).reshape(NW, NCH, K)
    dst = jnp.concatenate([edge_index[1], pad]).reshape(NW, NCH, K)
    xp = jnp.pad(x, ((0, NP - N), (0, 0)))
    b1r = b1.reshape(1, HID)
    asb1r = asb1.reshape(1, HID)
    w2p = jnp.pad(W2, ((0, 0), (0, HID - HID2)))
    b2p = jnp.pad(b2, ((0, HID - HID2),)).reshape(1, HID)
    asw2p = jnp.pad(asW2, ((0, HID - HID2), (0, HID - HID2)))
    asb2p = jnp.pad(asb2, ((0, HID - HID2),)).reshape(1, HID)
    gw2p = jnp.pad(gW2, ((0, HID - HID2), (0, HID - HID2)))
    wfp = jnp.pad(Wf, ((0, HID - HID2), (0, 128 - OUT)))
    bfp = jnp.pad(bf, ((0, 128 - OUT),)).reshape(1, 128)

    degp = _deg_kernel(dst)

    h = _k1a(xp, W1, b1r)
    ms, dinv = _k1b(h, gW1, degp)
    for _ in range(2):
        accp = _agg128(ms, src, dst)
        h, ms = _k2(h, accp, dinv, asW1, asb1r, gW1)
    accp = _agg128(ms, src, dst)
    h, ms = _k4(h, accp, dinv, asW1, asb1r, gW1, w2p, b2p, gw2p)
    accp = _agg128(ms, src, dst)
    out = _k5(h, accp, dinv, asw2p, asb2p, gw2p, wfp, bfp)
    return out[:N, :OUT]


# speedup vs baseline: 1.3663x; 1.0604x over previous
import functools

import jax
import jax.numpy as jnp
from jax import lax
from jax.experimental import pallas as pl
from jax.experimental.pallas import tpu as pltpu
from jax.experimental.pallas import tpu_sc as plsc

N = 10000
E = 320000
NP = 10240
IN_DIM = 128
HID = 128
HID2 = 64
OUT = 7

NC = 2
NS = 16
NW = NC * NS
K = 80
NCH = 128
EPAD = NW * NCH * K
RPT = NP // NS
ZR = 160

EPS = 0.1
GAMMA = 0.1
NEG_SLOPE = 0.01

_mesh = functools.partial(
    plsc.VectorSubcoreMesh, core_axis_name="c", subcore_axis_name="s",
    num_cores=NC, num_subcores=NS)


def _zero_vmem(buf, rows, cols):
    lanes = cols // 16

    def body(i, carry):
        buf[i // lanes, pl.ds((i % lanes) * 16, 16)] = jnp.zeros((16,), jnp.float32)
        return carry

    lax.fori_loop(0, rows * lanes, body, 0)


def _make_deg_kernel():
    D = 16

    @functools.partial(
        pl.kernel,
        out_type=jax.ShapeDtypeStruct((NC, NP, D), jnp.float32),
        mesh=_mesh(),
        scratch_types=[
            pltpu.VMEM((K,), jnp.int32),
            pltpu.VMEM((K,), jnp.int32),
            pltpu.VMEM((K, D), jnp.float32),
            pltpu.VMEM((K, D), jnp.float32),
            pltpu.VMEM_SHARED((NP, D), jnp.float32),
            pltpu.SemaphoreType.DMA,
            pltpu.SemaphoreType.DMA,
        ],
    )
    def deg_kernel(dst_hbm, out_hbm, dbuf0, dbuf1, ones, zbuf, acc,
                   semd0, semd1):
        cid = lax.axis_index("c")
        sid = lax.axis_index("s")
        wid = sid * NC + cid

        def ones_body(i, carry):
            ones[i, pl.ds(0, 16)] = jnp.ones((16,), jnp.float32)
            return carry

        lax.fori_loop(0, K, ones_body, 0)

        _zero_vmem(zbuf, K, D)
        base_r = sid * RPT
        for b in range(RPT // K):
            pltpu.sync_copy(zbuf, acc.at[pl.ds(base_r + b * K, K)])
        plsc.subcore_barrier()

        def d_start(ch, dbuf, sem):
            pltpu.make_async_copy(dst_hbm.at[wid, ch], dbuf, sem).start()

        def d_wait(ch, dbuf, sem):
            pltpu.make_async_copy(dst_hbm.at[wid, ch], dbuf, sem).wait()

        d_start(0, dbuf0, semd0)

        def body(i, carry):
            ch0 = 2 * i
            ch1 = 2 * i + 1
            d_start(ch1, dbuf1, semd1)
            d_wait(ch0, dbuf0, semd0)
            pltpu.sync_copy(ones, acc.at[dbuf0], add=True)

            @pl.when(ch1 + 1 < NCH)
            def _():
                d_start(ch1 + 1, dbuf0, semd0)

            d_wait(ch1, dbuf1, semd1)
            pltpu.sync_copy(ones, acc.at[dbuf1], add=True)
            return carry

        lax.fori_loop(0, NCH // 2, body, 0)
        if NCH % 2:
            d_wait(NCH - 1, dbuf0, semd0)
            pltpu.sync_copy(ones, acc.at[dbuf0], add=True)
        plsc.subcore_barrier()

        for b in range(RPT // K):
            r = base_r + b * K
            pltpu.sync_copy(acc.at[pl.ds(r, K)], zbuf)
            pltpu.sync_copy(zbuf, out_hbm.at[cid, pl.ds(r, K)])

    return deg_kernel


def _make_agg_kernel(D):
    @functools.partial(
        pl.kernel,
        out_type=jax.ShapeDtypeStruct((NC, NP, D), jnp.float32),
        mesh=_mesh(),
        scratch_types=[
            [pltpu.VMEM((K,), jnp.int32)] * 6,
            [pltpu.VMEM((K,), jnp.int32)] * 6,
            [pltpu.VMEM((K, D), jnp.float32)] * 3,
            pltpu.VMEM_SHARED((NP, D), jnp.float32),
            [pltpu.SemaphoreType.DMA] * 3,
            [pltpu.SemaphoreType.DMA] * 3,
            [pltpu.SemaphoreType.DMA] * 6,
            [pltpu.SemaphoreType.DMA] * 6,
        ],
    )
    def agg_kernel(ms_hbm, src_hbm, dst_hbm, out_hbm,
                   sbuf, dbuf, rows, acc, semg, sems, semi, semd):
        cid = lax.axis_index("c")
        sid = lax.axis_index("s")
        wid = sid * NC + cid
        rows0, rows1 = rows[0], rows[1]
        sem0, sem1 = semg[0], semg[1]

        _zero_vmem(rows0, K, D)
        base_r = sid * RPT
        for b in range(RPT // K):
            pltpu.sync_copy(rows0, acc.at[pl.ds(base_r + b * K, K)])
        plsc.subcore_barrier()

        def i_start(gch, m):
            pltpu.make_async_copy(src_hbm.at[wid, gch], sbuf[m], semi[m]).start()

        def i_wait(gch, m):
            pltpu.make_async_copy(src_hbm.at[wid, gch], sbuf[m], semi[m]).wait()

        def d_start(gch, m):
            pltpu.make_async_copy(dst_hbm.at[wid, gch], dbuf[m], semd[m]).start()

        def d_wait(gch, m):
            pltpu.make_async_copy(dst_hbm.at[wid, gch], dbuf[m], semd[m]).wait()

        def g_start(m, u):
            pltpu.make_async_copy(ms_hbm.at[sbuf[m]], rows[u], semg[u]).start()

        def g_wait(m, u):
            pltpu.make_async_copy(ms_hbm.at[sbuf[m]], rows[u], semg[u]).wait()

        def scat_start(u, m):
            pltpu.make_async_copy(rows[u], acc.at[dbuf[m]], sems[u]).start()

        def scat_wait(u, m):
            pltpu.make_async_copy(rows[u], acc.at[dbuf[m]], sems[u]).wait()

        for c in range(4):
            i_start(c, c)
            d_start(c, c)
        i_wait(0, 0)
        g_start(0, 0)
        i_wait(1, 1)
        g_start(1, 1)

        def step(t, j, static=False):
            def maybe(cond, fn):
                if static:
                    if cond:
                        fn()
                else:
                    pl.when(cond)(fn)

            u = j % 3
            g_wait(j, u)
            d_wait(t, j)
            scat_start(u, j)

            j1 = (j + 5) % 6
            maybe((t >= 1) & (t + 2 < NCH), lambda: scat_wait(j1 % 3, j1))

            j2 = (j + 2) % 6

            def adv_gather():
                i_wait(t + 2, j2)
                g_start(j2, j2 % 3)

            maybe(t + 2 < NCH, adv_gather)

            j4 = (j + 4) % 6

            def adv_idx():
                i_start(t + 4, j4)
                d_start(t + 4, j4)

            maybe(t + 4 < NCH, adv_idx)

        def body(i, carry):
            for j in range(6):
                step(6 * i + j, j)
            return carry

        NB = (NCH - 2) // 6
        lax.fori_loop(0, NB, body, 0)
        for t in range(6 * NB, NCH):
            step(t, t % 6, static=True)
        for c in range(NCH - 3, NCH):
            scat_wait(c % 3, c % 6)
        plsc.subcore_barrier()

        nwo = RPT // K
        for b in range(nwo):
            buf = rows0 if b % 2 == 0 else rows1
            sem = sem0 if b % 2 == 0 else sem1
            if b >= 2:
                r_prev = base_r + (b - 2) * K
                pltpu.make_async_copy(
                    buf, out_hbm.at[cid, pl.ds(r_prev, K)], sem).wait()
            r = base_r + b * K
            pltpu.sync_copy(acc.at[pl.ds(r, K)], buf)
            pltpu.make_async_copy(buf, out_hbm.at[cid, pl.ds(r, K)], sem).start()
        for b in range(nwo - 2, nwo):
            buf = rows0 if b % 2 == 0 else rows1
            sem = sem0 if b % 2 == 0 else sem1
            r = base_r + b * K
            pltpu.make_async_copy(buf, out_hbm.at[cid, pl.ds(r, K)], sem).wait()

    return agg_kernel


_make_deg_kernel = functools.cache(_make_deg_kernel)
_make_agg_kernel = functools.cache(_make_agg_kernel)


def _deg_kernel(dst):
    return _make_deg_kernel()(dst)


def _agg128(ms, src, dst):
    return _make_agg_kernel(HID)(ms, src, dst)


BR = 1024
GRID = NP // BR


def _leaky(x):
    return jnp.where(x >= 0, x, NEG_SLOPE * x)


def _dinv_from(degp_ref):
    deg = degp_ref[0, :, 0:1] + degp_ref[1, :, 0:1] + 1.0
    return lax.rsqrt(deg)


def _dotT(a, w):
    return lax.dot_general(a, w, (((1,), (1,)), ((), ())),
                           preferred_element_type=jnp.float32)


def _dot(a, w):
    return lax.dot_general(a, w, (((1,), (0,)), ((), ())),
                           preferred_element_type=jnp.float32)


def _row_spec(d):
    return pl.BlockSpec((BR, d), lambda i: (i, 0))


def _full_spec(r, c):
    return pl.BlockSpec((r, c), lambda i: (0, 0))


_degp_spec = pl.BlockSpec((NC, BR, 16), lambda i: (0, i, 0))
_accp_spec128 = pl.BlockSpec((NC, BR, HID), lambda i: (0, i, 0))
_accp_spec64 = pl.BlockSpec((NC, BR, HID2), lambda i: (0, i, 0))


def _k1a_body(x_ref, w1_ref, b1_ref, h_ref):
    h_ref[...] = _leaky(_dot(x_ref[...], w1_ref[...]) + b1_ref[...])


def _k1b_body(h_ref, gw_ref, degp_ref, ms_ref, dinv_ref):
    dinv = _dinv_from(degp_ref)
    ms_ref[...] = _dot(h_ref[...], gw_ref[...]) * dinv
    dinv_ref[...] = dinv


def _iter_update(h, accp_ref, dinv, asw_ref, b_ref, gw_ref):
    m = _dot(h, gw_ref[...])
    acc = accp_ref[0] + accp_ref[1]
    g = dinv * acc + (dinv * dinv) * m
    z = _dotT(h, asw_ref[...]) - _dot(h, asw_ref[...]) - GAMMA * h + g + b_ref[...]
    return h + EPS * jnp.tanh(z)


def _k2_body(h_ref, accp_ref, dinv_ref, asw_ref, b_ref, gw_ref,
             h_out, ms_out):
    dinv = dinv_ref[...]
    h2 = _iter_update(h_ref[...], accp_ref, dinv, asw_ref, b_ref, gw_ref)
    h_out[...] = h2
    ms_out[...] = _dot(h2, gw_ref[...]) * dinv


def _k4_body(h_ref, accp_ref, dinv_ref, asw_ref, b_ref, gw_ref,
             w2_ref, b2_ref, gw2_ref,
             h_out, ms_out):
    dinv = dinv_ref[...]
    h2 = _iter_update(h_ref[...], accp_ref, dinv, asw_ref, b_ref, gw_ref)
    hb = _leaky(_dot(_leaky(h2), w2_ref[...]) + b2_ref[...])
    h_out[...] = hb
    ms_out[...] = _dot(hb, gw2_ref[...]) * dinv


def _k5_body(h_ref, accp_ref, dinv_ref, asw_ref, b_ref, gw_ref,
             wf_ref, bf_ref, out_ref):
    dinv = dinv_ref[...]
    h2 = _iter_update(h_ref[...], accp_ref, dinv, asw_ref, b_ref, gw_ref)
    logits = _dot(h2, wf_ref[...]) + bf_ref[...]
    col = lax.broadcasted_iota(jnp.int32, logits.shape, 1)
    z = jnp.where(col < OUT, logits, -1e30)
    zmax = jnp.max(z, axis=1, keepdims=True)
    lse = jnp.log(jnp.sum(jnp.exp(z - zmax), axis=1, keepdims=True)) + zmax
    out_ref[...] = (z - lse)[:, :8]


def _rows_out(d):
    return jax.ShapeDtypeStruct((NP, d), jnp.float32)


_dinv_spec = pl.BlockSpec((BR, 1), lambda i: (i, 0))

_k1a = pl.pallas_call(
    _k1a_body,
    grid=(GRID,),
    in_specs=[_row_spec(IN_DIM), _full_spec(IN_DIM, HID), _full_spec(1, HID)],
    out_specs=_row_spec(HID),
    out_shape=_rows_out(HID),
)

_k1b = pl.pallas_call(
    _k1b_body,
    grid=(GRID,),
    in_specs=[_row_spec(HID), _full_spec(HID, HID), _degp_spec],
    out_specs=[_row_spec(HID), _dinv_spec],
    out_shape=[_rows_out(HID), jax.ShapeDtypeStruct((NP, 1), jnp.float32)],
)

_k2 = pl.pallas_call(
    _k2_body,
    grid=(GRID,),
    in_specs=[_row_spec(HID), _accp_spec128, _dinv_spec,
              _full_spec(HID, HID), _full_spec(1, HID), _full_spec(HID, HID)],
    out_specs=[_row_spec(HID)] * 2,
    out_shape=[_rows_out(HID)] * 2,
)

_k4 = pl.pallas_call(
    _k4_body,
    grid=(GRID,),
    in_specs=[_row_spec(HID), _accp_spec128, _dinv_spec,
              _full_spec(HID, HID), _full_spec(1, HID), _full_spec(HID, HID),
              _full_spec(HID, HID), _full_spec(1, HID), _full_spec(HID, HID)],
    out_specs=[_row_spec(HID)] * 2,
    out_shape=[_rows_out(HID)] * 2,
)

_k5 = pl.pallas_call(
    _k5_body,
    grid=(GRID,),
    in_specs=[_row_spec(HID), _accp_spec128, _dinv_spec,
              _full_spec(HID, HID), _full_spec(1, HID), _full_spec(HID, HID),
              _full_spec(HID, 128), _full_spec(1, 128)],
    out_specs=pl.BlockSpec((BR, 8), lambda i: (i, 0)),
    out_shape=jax.ShapeDtypeStruct((NP, 8), jnp.float32),
)


def kernel(x, edge_index, W1, b1, asW1, asb1, gW1, W2, b2, asW2, asb2, gW2, Wf, bf):
    pad = (jnp.arange(EPAD - E, dtype=edge_index.dtype) % (NP - N)) + N
    src = jnp.concatenate([edge_index[0], pad]).reshape(NW, NCH, K)
    dst = jnp.concatenate([edge_index[1], pad]).reshape(NW, NCH, K)
    xp = jnp.pad(x, ((0, NP - N), (0, 0)))
    b1r = b1.reshape(1, HID)
    asb1r = asb1.reshape(1, HID)
    w2p = jnp.pad(W2, ((0, 0), (0, HID - HID2)))
    b2p = jnp.pad(b2, ((0, HID - HID2),)).reshape(1, HID)
    asw2p = jnp.pad(asW2, ((0, HID - HID2), (0, HID - HID2)))
    asb2p = jnp.pad(asb2, ((0, HID - HID2),)).reshape(1, HID)
    gw2p = jnp.pad(gW2, ((0, HID - HID2), (0, HID - HID2)))
    wfp = jnp.pad(Wf, ((0, HID - HID2), (0, 128 - OUT)))
    bfp = jnp.pad(bf, ((0, 128 - OUT),)).reshape(1, 128)

    degp = _deg_kernel(dst)

    h = _k1a(xp, W1, b1r)
    ms, dinv = _k1b(h, gW1, degp)
    for _ in range(2):
        accp = _agg128(ms, src, dst)
        h, ms = _k2(h, accp, dinv, asW1, asb1r, gW1)
    accp = _agg128(ms, src, dst)
    h, ms = _k4(h, accp, dinv, asW1, asb1r, gW1, w2p, b2p, gw2p)
    accp = _agg128(ms, src, dst)
    out = _k5(h, accp, dinv, asw2p, asb2p, gw2p, wfp, bfp)
    return out[:N, :OUT]


# speedup vs baseline: 1.3869x; 1.0151x over previous
import functools

import jax
import jax.numpy as jnp
from jax import lax
from jax.experimental import pallas as pl
from jax.experimental.pallas import tpu as pltpu
from jax.experimental.pallas import tpu_sc as plsc

N = 10000
E = 320000
NP = 10240
IN_DIM = 128
HID = 128
HID2 = 64
OUT = 7

NC = 2
NS = 16
NW = NC * NS
K = 80
NCH = 128
EPAD = NW * NCH * K
RPT = NP // NS
ZR = 160

EPS = 0.1
GAMMA = 0.1
NEG_SLOPE = 0.01

_mesh = functools.partial(
    plsc.VectorSubcoreMesh, core_axis_name="c", subcore_axis_name="s",
    num_cores=NC, num_subcores=NS)


def _zero_vmem(buf, rows, cols):
    lanes = cols // 16

    def body(i, carry):
        buf[i // lanes, pl.ds((i % lanes) * 16, 16)] = jnp.zeros((16,), jnp.float32)
        return carry

    lax.fori_loop(0, rows * lanes, body, 0)


def _make_deg_kernel():
    D = 16

    @functools.partial(
        pl.kernel,
        out_type=jax.ShapeDtypeStruct((NC, NP, D), jnp.float32),
        mesh=_mesh(),
        scratch_types=[
            pltpu.VMEM((K,), jnp.int32),
            pltpu.VMEM((K,), jnp.int32),
            pltpu.VMEM((K, D), jnp.float32),
            pltpu.VMEM((K, D), jnp.float32),
            pltpu.VMEM_SHARED((NP, D), jnp.float32),
            pltpu.SemaphoreType.DMA,
            pltpu.SemaphoreType.DMA,
        ],
    )
    def deg_kernel(dst_hbm, out_hbm, dbuf0, dbuf1, ones, zbuf, acc,
                   semd0, semd1):
        cid = lax.axis_index("c")
        sid = lax.axis_index("s")
        wid = sid * NC + cid

        def ones_body(i, carry):
            ones[i, pl.ds(0, 16)] = jnp.ones((16,), jnp.float32)
            return carry

        lax.fori_loop(0, K, ones_body, 0)

        _zero_vmem(zbuf, K, D)
        base_r = sid * RPT
        for b in range(RPT // K):
            pltpu.sync_copy(zbuf, acc.at[pl.ds(base_r + b * K, K)])
        plsc.subcore_barrier()

        def d_start(ch, dbuf, sem):
            pltpu.make_async_copy(dst_hbm.at[wid, ch], dbuf, sem).start()

        def d_wait(ch, dbuf, sem):
            pltpu.make_async_copy(dst_hbm.at[wid, ch], dbuf, sem).wait()

        d_start(0, dbuf0, semd0)

        def body(i, carry):
            ch0 = 2 * i
            ch1 = 2 * i + 1
            d_start(ch1, dbuf1, semd1)
            d_wait(ch0, dbuf0, semd0)
            pltpu.sync_copy(ones, acc.at[dbuf0], add=True)

            @pl.when(ch1 + 1 < NCH)
            def _():
                d_start(ch1 + 1, dbuf0, semd0)

            d_wait(ch1, dbuf1, semd1)
            pltpu.sync_copy(ones, acc.at[dbuf1], add=True)
            return carry

        lax.fori_loop(0, NCH // 2, body, 0)
        if NCH % 2:
            d_wait(NCH - 1, dbuf0, semd0)
            pltpu.sync_copy(ones, acc.at[dbuf0], add=True)
        plsc.subcore_barrier()

        for b in range(RPT // K):
            r = base_r + b * K
            pltpu.sync_copy(acc.at[pl.ds(r, K)], zbuf)
            pltpu.sync_copy(zbuf, out_hbm.at[cid, pl.ds(r, K)])

    return deg_kernel


def _make_agg_kernel(D):
    @functools.partial(
        pl.kernel,
        out_type=jax.ShapeDtypeStruct((NC, NP, D), jnp.float32),
        mesh=_mesh(),
        scratch_types=[
            [pltpu.VMEM((K,), jnp.int32)] * 6,
            [pltpu.VMEM((K,), jnp.int32)] * 6,
            [pltpu.VMEM((K, D), jnp.float32)] * 3,
            pltpu.VMEM_SHARED((NP, D), jnp.float32),
            [pltpu.SemaphoreType.DMA] * 3,
            [pltpu.SemaphoreType.DMA] * 3,
            [pltpu.SemaphoreType.DMA] * 6,
            [pltpu.SemaphoreType.DMA] * 6,
        ],
    )
    def agg_kernel(ms_hbm, src_hbm, dst_hbm, out_hbm,
                   sbuf, dbuf, rows, acc, semg, sems, semi, semd):
        cid = lax.axis_index("c")
        sid = lax.axis_index("s")
        wid = sid * NC + cid
        rows0, rows1 = rows[0], rows[1]
        sem0, sem1 = semg[0], semg[1]

        _zero_vmem(rows0, K, D)
        base_r = sid * RPT
        for b in range(RPT // K):
            pltpu.sync_copy(rows0, acc.at[pl.ds(base_r + b * K, K)])
        plsc.subcore_barrier()

        def i_start(gch, m):
            pltpu.make_async_copy(src_hbm.at[wid, gch], sbuf[m], semi[m]).start()

        def i_wait(gch, m):
            pltpu.make_async_copy(src_hbm.at[wid, gch], sbuf[m], semi[m]).wait()

        def d_start(gch, m):
            pltpu.make_async_copy(dst_hbm.at[wid, gch], dbuf[m], semd[m]).start()

        def d_wait(gch, m):
            pltpu.make_async_copy(dst_hbm.at[wid, gch], dbuf[m], semd[m]).wait()

        def g_start(m, u):
            pltpu.make_async_copy(ms_hbm.at[sbuf[m]], rows[u], semg[u]).start()

        def g_wait(m, u):
            pltpu.make_async_copy(ms_hbm.at[sbuf[m]], rows[u], semg[u]).wait()

        def scat_start(u, m):
            pltpu.make_async_copy(rows[u], acc.at[dbuf[m]], sems[u]).start()

        def scat_wait(u, m):
            pltpu.make_async_copy(rows[u], acc.at[dbuf[m]], sems[u]).wait()

        for c in range(4):
            i_start(c, c)
            d_start(c, c)
        i_wait(0, 0)
        g_start(0, 0)
        i_wait(1, 1)
        g_start(1, 1)

        def step(t, j, static=False):
            def maybe(cond, fn):
                if static:
                    if cond:
                        fn()
                else:
                    pl.when(cond)(fn)

            u = j % 3
            g_wait(j, u)
            d_wait(t, j)
            scat_start(u, j)

            j1 = (j + 5) % 6
            maybe((t >= 1) & (t + 2 < NCH), lambda: scat_wait(j1 % 3, j1))

            j2 = (j + 2) % 6

            def adv_gather():
                i_wait(t + 2, j2)
                g_start(j2, j2 % 3)

            maybe(t + 2 < NCH, adv_gather)

            j4 = (j + 4) % 6

            def adv_idx():
                i_start(t + 4, j4)
                d_start(t + 4, j4)

            maybe(t + 4 < NCH, adv_idx)

        def body(i, carry):
            for j in range(6):
                step(6 * i + j, j)
            return carry

        NB = (NCH - 2) // 6
        lax.fori_loop(0, NB, body, 0)
        for t in range(6 * NB, NCH):
            step(t, t % 6, static=True)
        for c in range(NCH - 3, NCH):
            scat_wait(c % 3, c % 6)
        plsc.subcore_barrier()

        nwo = RPT // K
        for b in range(nwo):
            buf = rows0 if b % 2 == 0 else rows1
            sem = sem0 if b % 2 == 0 else sem1
            if b >= 2:
                r_prev = base_r + (b - 2) * K
                pltpu.make_async_copy(
                    buf, out_hbm.at[cid, pl.ds(r_prev, K)], sem).wait()
            r = base_r + b * K
            pltpu.sync_copy(acc.at[pl.ds(r, K)], buf)
            pltpu.make_async_copy(buf, out_hbm.at[cid, pl.ds(r, K)], sem).start()
        for b in range(nwo - 2, nwo):
            buf = rows0 if b % 2 == 0 else rows1
            sem = sem0 if b % 2 == 0 else sem1
            r = base_r + b * K
            pltpu.make_async_copy(buf, out_hbm.at[cid, pl.ds(r, K)], sem).wait()

    return agg_kernel


_make_deg_kernel = functools.cache(_make_deg_kernel)
_make_agg_kernel = functools.cache(_make_agg_kernel)


def _deg_kernel(dst):
    return _make_deg_kernel()(dst)


def _agg128(ms, src, dst):
    return _make_agg_kernel(HID)(ms, src, dst)


BR = 2048
GRID = NP // BR


def _leaky(x):
    return jnp.where(x >= 0, x, NEG_SLOPE * x)


def _dinv_from(degp_ref):
    deg = degp_ref[0, :, 0:1] + degp_ref[1, :, 0:1] + 1.0
    return lax.rsqrt(deg)


def _dotT(a, w):
    return lax.dot_general(a, w, (((1,), (1,)), ((), ())),
                           preferred_element_type=jnp.float32)


def _dot(a, w):
    return lax.dot_general(a, w, (((1,), (0,)), ((), ())),
                           preferred_element_type=jnp.float32)


def _row_spec(d):
    return pl.BlockSpec((BR, d), lambda i: (i, 0))


def _full_spec(r, c):
    return pl.BlockSpec((r, c), lambda i: (0, 0))


_degp_spec = pl.BlockSpec((NC, BR, 16), lambda i: (0, i, 0))
_accp_spec128 = pl.BlockSpec((NC, BR, HID), lambda i: (0, i, 0))
_accp_spec64 = pl.BlockSpec((NC, BR, HID2), lambda i: (0, i, 0))


def _k1a_body(x_ref, w1_ref, b1_ref, h_ref):
    h_ref[...] = _leaky(_dot(x_ref[...], w1_ref[...]) + b1_ref[...])


def _k1b_body(h_ref, gw_ref, degp_ref, ms_ref, dinv_ref):
    dinv = _dinv_from(degp_ref)
    ms_ref[...] = _dot(h_ref[...], gw_ref[...]) * dinv
    dinv_ref[...] = dinv


def _iter_update(h, accp_ref, dinv, asw_ref, b_ref, gw_ref):
    m = _dot(h, gw_ref[...])
    acc = accp_ref[0] + accp_ref[1]
    g = dinv * acc + (dinv * dinv) * m
    z = _dotT(h, asw_ref[...]) - _dot(h, asw_ref[...]) - GAMMA * h + g + b_ref[...]
    return h + EPS * jnp.tanh(z)


def _k2_body(h_ref, accp_ref, dinv_ref, asw_ref, b_ref, gw_ref,
             h_out, ms_out):
    dinv = dinv_ref[...]
    h2 = _iter_update(h_ref[...], accp_ref, dinv, asw_ref, b_ref, gw_ref)
    h_out[...] = h2
    ms_out[...] = _dot(h2, gw_ref[...]) * dinv


def _k4_body(h_ref, accp_ref, dinv_ref, asw_ref, b_ref, gw_ref,
             w2_ref, b2_ref, gw2_ref,
             h_out, ms_out):
    dinv = dinv_ref[...]
    h2 = _iter_update(h_ref[...], accp_ref, dinv, asw_ref, b_ref, gw_ref)
    hb = _leaky(_dot(_leaky(h2), w2_ref[...]) + b2_ref[...])
    h_out[...] = hb
    ms_out[...] = _dot(hb, gw2_ref[...]) * dinv


def _k5_body(h_ref, accp_ref, dinv_ref, asw_ref, b_ref, gw_ref,
             wf_ref, bf_ref, out_ref):
    dinv = dinv_ref[...]
    h2 = _iter_update(h_ref[...], accp_ref, dinv, asw_ref, b_ref, gw_ref)
    logits = _dot(h2, wf_ref[...]) + bf_ref[...]
    col = lax.broadcasted_iota(jnp.int32, logits.shape, 1)
    z = jnp.where(col < OUT, logits, -1e30)
    zmax = jnp.max(z, axis=1, keepdims=True)
    lse = jnp.log(jnp.sum(jnp.exp(z - zmax), axis=1, keepdims=True)) + zmax
    out_ref[...] = (z - lse)[:, :8]


def _rows_out(d):
    return jax.ShapeDtypeStruct((NP, d), jnp.float32)


_dinv_spec = pl.BlockSpec((BR, 1), lambda i: (i, 0))

_k1a = pl.pallas_call(
    _k1a_body,
    grid=(GRID,),
    in_specs=[_row_spec(IN_DIM), _full_spec(IN_DIM, HID), _full_spec(1, HID)],
    out_specs=_row_spec(HID),
    out_shape=_rows_out(HID),
)

_k1b = pl.pallas_call(
    _k1b_body,
    grid=(GRID,),
    in_specs=[_row_spec(HID), _full_spec(HID, HID), _degp_spec],
    out_specs=[_row_spec(HID), _dinv_spec],
    out_shape=[_rows_out(HID), jax.ShapeDtypeStruct((NP, 1), jnp.float32)],
)

_k2 = pl.pallas_call(
    _k2_body,
    grid=(GRID,),
    in_specs=[_row_spec(HID), _accp_spec128, _dinv_spec,
              _full_spec(HID, HID), _full_spec(1, HID), _full_spec(HID, HID)],
    out_specs=[_row_spec(HID)] * 2,
    out_shape=[_rows_out(HID)] * 2,
)

_k4 = pl.pallas_call(
    _k4_body,
    grid=(GRID,),
    in_specs=[_row_spec(HID), _accp_spec128, _dinv_spec,
              _full_spec(HID, HID), _full_spec(1, HID), _full_spec(HID, HID),
              _full_spec(HID, HID), _full_spec(1, HID), _full_spec(HID, HID)],
    out_specs=[_row_spec(HID)] * 2,
    out_shape=[_rows_out(HID)] * 2,
)

_k5 = pl.pallas_call(
    _k5_body,
    grid=(GRID,),
    in_specs=[_row_spec(HID), _accp_spec128, _dinv_spec,
              _full_spec(HID, HID), _full_spec(1, HID), _full_spec(HID, HID),
              _full_spec(HID, 128), _full_spec(1, 128)],
    out_specs=pl.BlockSpec((BR, 8), lambda i: (i, 0)),
    out_shape=jax.ShapeDtypeStruct((NP, 8), jnp.float32),
)


def kernel(x, edge_index, W1, b1, asW1, asb1, gW1, W2, b2, asW2, asb2, gW2, Wf, bf):
    pad = (jnp.arange(EPAD - E, dtype=edge_index.dtype) % (NP - N)) + N
    src = jnp.concatenate([edge_index[0], pad]).reshape(NW, NCH, K)
    dst = jnp.concatenate([edge_index[1], pad]).reshape(NW, NCH, K)
    xp = jnp.pad(x, ((0, NP - N), (0, 0)))
    b1r = b1.reshape(1, HID)
    asb1r = asb1.reshape(1, HID)
    w2p = jnp.pad(W2, ((0, 0), (0, HID - HID2)))
    b2p = jnp.pad(b2, ((0, HID - HID2),)).reshape(1, HID)
    asw2p = jnp.pad(asW2, ((0, HID - HID2), (0, HID - HID2)))
    asb2p = jnp.pad(asb2, ((0, HID - HID2),)).reshape(1, HID)
    gw2p = jnp.pad(gW2, ((0, HID - HID2), (0, HID - HID2)))
    wfp = jnp.pad(Wf, ((0, HID - HID2), (0, 128 - OUT)))
    bfp = jnp.pad(bf, ((0, 128 - OUT),)).reshape(1, 128)

    degp = _deg_kernel(dst)

    h = _k1a(xp, W1, b1r)
    ms, dinv = _k1b(h, gW1, degp)
    for _ in range(2):
        accp = _agg128(ms, src, dst)
        h, ms = _k2(h, accp, dinv, asW1, asb1r, gW1)
    accp = _agg128(ms, src, dst)
    h, ms = _k4(h, accp, dinv, asW1, asb1r, gW1, w2p, b2p, gw2p)
    accp = _agg128(ms, src, dst)
    out = _k5(h, accp, dinv, asw2p, asb2p, gw2p, wfp, bfp)
    return out[:N, :OUT]
